# baseline (speedup 1.0000x reference)
import functools

import jax
import jax.numpy as jnp
from jax import lax
from jax.experimental import pallas as pl
from jax.experimental.pallas import tpu as pltpu
from jax.experimental.pallas import tpu_sc as plsc

PAD = 1
HID = 768
EPS = 1e-12

_GATHER_DNUMS = lax.GatherDimensionNumbers(
    offset_dims=(), collapsed_slice_dims=(0,), start_index_map=(0,))


def _shuffle(v, idx):
    return lax.gather(v, idx[:, None], _GATHER_DNUMS, (1,),
                      mode=lax.GatherScatterMode.PROMISE_IN_BOUNDS)

NC = 2
NS = 16
NW = NC * NS
LANES = 16
NVH = HID // LANES


def _posid_body(ids_ref, out_ref):
    ids = ids_ref[...]
    m = (ids != PAD).astype(jnp.int32)
    acc = m
    s = ids.shape[1]
    k = 1
    while k < s:
        shifted = jnp.concatenate(
            [jnp.zeros(ids.shape[:1] + (k,), jnp.int32), acc[:, :-k]], axis=1
        )
        acc = acc + shifted
        k *= 2
    out_ref[...] = acc * m + PAD


def _sc_body(tpw, chunk, word_hbm, pos_hbm, item_hbm, idw_hbm, idp_hbm,
             idi_hbm, lnw_hbm, lnb_hbm, out_hbm,
             idxw_v, idxp_v, idxi_v, bw_v, bp_v, bi_v, wv, bv, sem):
    wid = lax.axis_index("s") * NC + lax.axis_index("c")
    base = wid * tpw
    pltpu.sync_copy(lnw_hbm, wv)
    pltpu.sync_copy(lnb_hbm, bv)

    half = jnp.float32(0.5)
    three_half = jnp.float32(1.5)
    magic = jnp.int32(0x5F3759DF)
    lane = lax.iota(jnp.int32, LANES)
    perms = [lane ^ k for k in (1, 2, 4, 8)]

    def do_chunk(g, _):
        off = base + g * chunk
        pltpu.sync_copy(idw_hbm.at[pl.ds(off, chunk)], idxw_v)
        pltpu.sync_copy(idp_hbm.at[pl.ds(off, chunk)], idxp_v)
        pltpu.sync_copy(idi_hbm.at[pl.ds(off, chunk)], idxi_v)
        cw = pltpu.async_copy(word_hbm.at[idxw_v], bw_v, sem)
        cp = pltpu.async_copy(pos_hbm.at[idxp_v], bp_v, sem)
        ci = pltpu.async_copy(item_hbm.at[idxi_v], bi_v, sem)
        cw.wait()
        cp.wait()
        ci.wait()

        def ln_token(t, _):
            s = jnp.zeros((LANES,), jnp.float32)
            ss = jnp.zeros((LANES,), jnp.float32)
            for i in range(NVH):
                sl = pl.ds(i * LANES, LANES)
                x = bw_v[t, sl] + bp_v[t, sl] + bi_v[t, sl]
                bw_v[t, sl] = x
                s = s + x
                ss = ss + x * x
            for p in perms:
                s = s + _shuffle(s, p)
                ss = ss + _shuffle(ss, p)
            mu_v = s * (1.0 / HID)
            vv = ss * (1.0 / HID) - mu_v * mu_v + EPS
            bits = lax.bitcast_convert_type(vv, jnp.int32)
            y = lax.bitcast_convert_type(magic - (bits >> 1), jnp.float32)
            for _it in range(3):
                y = y * (three_half - half * vv * y * y)
            for i in range(NVH):
                sl = pl.ds(i * LANES, LANES)
                x = bw_v[t, sl]
                bw_v[t, sl] = (x - mu_v) * y * wv[sl] + bv[sl]
            return 0

        lax.fori_loop(0, chunk, ln_token, 0)
        pltpu.sync_copy(bw_v, out_hbm.at[pl.ds(off, chunk)])
        return 0

    lax.fori_loop(0, tpw // chunk, do_chunk, 0)


def kernel(input_ids, item_position_ids, word_embeddings, position_embeddings,
           item_position_embeddings, ln_weight, ln_bias):
    b, s = input_ids.shape
    n = b * s
    tpw = n // NW
    chunk = 32

    position_ids = pl.pallas_call(
        _posid_body,
        out_shape=jax.ShapeDtypeStruct((b, s), jnp.int32),
    )(input_ids.astype(jnp.int32))

    mesh = plsc.VectorSubcoreMesh(core_axis_name="c", subcore_axis_name="s")
    sc = pl.kernel(
        functools.partial(_sc_body, tpw, chunk),
        out_type=jax.ShapeDtypeStruct((n, HID), jnp.float32),
        mesh=mesh,
        scratch_types=[
            pltpu.VMEM((chunk,), jnp.int32),
            pltpu.VMEM((chunk,), jnp.int32),
            pltpu.VMEM((chunk,), jnp.int32),
            pltpu.VMEM((chunk, HID), jnp.float32),
            pltpu.VMEM((chunk, HID), jnp.float32),
            pltpu.VMEM((chunk, HID), jnp.float32),
            pltpu.VMEM((HID,), jnp.float32),
            pltpu.VMEM((HID,), jnp.float32),
            pltpu.SemaphoreType.DMA,
        ],
    )
    out = sc(
        word_embeddings,
        position_embeddings,
        item_position_embeddings,
        input_ids.reshape(n).astype(jnp.int32),
        position_ids.reshape(n),
        item_position_ids.reshape(n).astype(jnp.int32),
        ln_weight,
        ln_bias,
    )
    return out.reshape(b, s, HID)
